# baseline (speedup 1.0000x reference)
import functools

import jax
import jax.numpy as jnp
from jax import lax
from jax.experimental import pallas as pl
from jax.experimental.pallas import tpu as pltpu
from jax.experimental.pallas import tpu_sc as plsc

N = 10000
E = 320000
D_IN = 128
D_EDGE = 16
HID = 128
D_OUT = 128

MLP_BLOCK = 1280

NC = 2
NS = 16
NW = NC * NS

NCH = 5
EC = E // NCH
EPW = EC // NW
KG = 80
NITER = EPW // KG
NB = 5
NGRP = NITER // NB
NB_S = 4
NGRP_S = NITER // NB_S
NP = 10240
RPT = NP // NS
HROWS = 5
HR = 2048

_SC_MESH = plsc.VectorSubcoreMesh(
    core_axis_name="c", subcore_axis_name="s", num_cores=NC, num_subcores=NS
)
_SC_PARAMS = pltpu.CompilerParams(needs_layout_passes=False)


def _sc_gather_body(xp_hbm, dst_hbm, e1_hbm, cnt_hbm, idx_v, rows, hist_v,
                    sem_i, sems_g, sems_w):
    c = lax.axis_index("c")
    s = lax.axis_index("s")
    wid = c * NS + s
    base = wid * EPW

    idx_load = pltpu.async_copy(dst_hbm.at[wid], idx_v, sem_i)

    zeros16 = jnp.zeros((16,), jnp.float32)

    def zero_hist(i, carry):
        for r in range(HROWS):
            hist_v[r, pl.ds(i * 16, 16)] = zeros16
        return carry

    lax.fori_loop(0, HR // 16, zero_hist, 0)
    idx_load.wait()

    ones = jnp.ones((16,), jnp.float32)

    def hist_update(i):
        for j in range(KG // 16):
            idxs = idx_v[i, pl.ds(j * 16, 16)]
            plsc.addupdate_scatter(
                hist_v, [lax.shift_right_logical(idxs, 11),
                         lax.bitwise_and(idxs, HR - 1)], ones)

    def group(g, carry):
        descs = []
        for b in range(NB):
            i = g * NB + b

            @pl.when(g > 0)
            def _():
                pltpu.make_async_copy(
                    rows[b], e1_hbm.at[pl.ds(base + i * KG, KG)], sems_w[b]
                ).wait()

            descs.append(
                pltpu.async_copy(xp_hbm.at[idx_v.at[i]], rows[b], sems_g[b])
            )
        for b in range(NB):
            i = g * NB + b
            hist_update(i)
            descs[b].wait()
            pltpu.async_copy(rows[b], e1_hbm.at[pl.ds(base + i * KG, KG)],
                             sems_w[b])
        return carry

    lax.fori_loop(0, NGRP, group, 0)

    for b in range(NB):
        pltpu.make_async_copy(
            rows[b], e1_hbm.at[pl.ds(base, KG)], sems_w[b]
        ).wait()
    pltpu.sync_copy(hist_v, cnt_hbm.at[wid])


@functools.partial(
    pl.kernel,
    out_type=(
        jax.ShapeDtypeStruct((EC, HID), jnp.float32),
        jax.ShapeDtypeStruct((NW, HROWS, HR), jnp.float32),
    ),
    mesh=_SC_MESH,
    scratch_types=[
        pltpu.VMEM((NITER, KG), jnp.int32),
        [pltpu.VMEM((KG, HID), jnp.float32) for _ in range(NB)],
        pltpu.VMEM((HROWS, HR), jnp.float32),
        pltpu.SemaphoreType.DMA,
        [pltpu.SemaphoreType.DMA for _ in range(NB)],
        [pltpu.SemaphoreType.DMA for _ in range(NB)],
    ],
    compiler_params=_SC_PARAMS,
)
def _sc_gather(*refs):
    _sc_gather_body(*refs)


def _make_scatter(nm):
    def body(*refs):
        ms = refs[:nm]
        dsts = refs[nm:2 * nm]
        z_hbm = refs[2 * nm]
        sums_hbm = refs[2 * nm + 1]
        idx_v = refs[2 * nm + 2]
        rows = refs[2 * nm + 3]
        sem_i = refs[2 * nm + 4]
        sems_l = refs[2 * nm + 5]
        sems_s = refs[2 * nm + 6]
        acc_sh = refs[2 * nm + 7]

        c = lax.axis_index("c")
        s = lax.axis_index("s")
        wid = c * NS + s
        base = wid * EPW

        pltpu.sync_copy(z_hbm, rows[0])
        for k in range(RPT // KG):
            r0 = s * RPT + k * KG
            pltpu.sync_copy(rows[0], acc_sh.at[pl.ds(r0, KG)])
        plsc.subcore_barrier()

        for m_hbm, dst_hbm in zip(ms, dsts):
            pltpu.async_copy(dst_hbm.at[wid], idx_v, sem_i).wait()

            def group(g, carry):
                for b in range(NB_S):
                    i = g * NB_S + b

                    @pl.when(g > 0)
                    def _():
                        pltpu.make_async_copy(
                            rows[b], acc_sh.at[idx_v.at[i]], sems_s[b]
                        ).wait()

                    pltpu.async_copy(m_hbm.at[pl.ds(base + i * KG, KG)],
                                     rows[b], sems_l[b])
                for b in range(NB_S):
                    i = g * NB_S + b
                    pltpu.make_async_copy(
                        m_hbm.at[pl.ds(base, KG)], rows[b], sems_l[b]).wait()
                    pltpu.async_copy(rows[b], acc_sh.at[idx_v.at[i]],
                                     sems_s[b], add=True)
                return carry

            lax.fori_loop(0, NGRP_S, group, 0)

            for b in range(NB_S):
                pltpu.make_async_copy(
                    rows[b], acc_sh.at[idx_v.at[0]], sems_s[b]).wait()

            i = NGRP_S * NB_S
            pltpu.sync_copy(m_hbm.at[pl.ds(base + i * KG, KG)], rows[0])
            pltpu.sync_copy(rows[0], acc_sh.at[idx_v.at[i]], add=True)

        plsc.subcore_barrier()

        for k in range(RPT // KG):
            r0 = s * RPT + k * KG
            pltpu.sync_copy(acc_sh.at[pl.ds(r0, KG)], rows[0])
            pltpu.sync_copy(rows[0], sums_hbm.at[c, pl.ds(r0, KG)])

    return functools.partial(
        pl.kernel,
        out_type=jax.ShapeDtypeStruct((NC, NP, HID), jnp.float32),
        mesh=_SC_MESH,
        scratch_types=[
            pltpu.VMEM((NITER, KG), jnp.int32),
            [pltpu.VMEM((KG, HID), jnp.float32) for _ in range(NB_S)],
            pltpu.SemaphoreType.DMA,
            [pltpu.SemaphoreType.DMA for _ in range(NB_S)],
            [pltpu.SemaphoreType.DMA for _ in range(NB_S)],
            pltpu.VMEM_SHARED((NP, HID), jnp.float32),
        ],
        compiler_params=_SC_PARAMS,
    )(body)


_sc_scatter2 = _make_scatter(2)
_sc_scatter3 = _make_scatter(3)


def _xp_body(x_ref, w_ref, b_ref, o_ref):
    o_ref[...] = (
        jnp.dot(x_ref[...], w_ref[...], preferred_element_type=jnp.float32)
        + b_ref[...]
    )


def _node_precompute(x, W1a, b1):
    grid = (10,)
    return pl.pallas_call(
        _xp_body,
        grid=grid,
        in_specs=[
            pl.BlockSpec((N // 10, D_IN), lambda i: (i, 0)),
            pl.BlockSpec((D_IN, HID), lambda i: (0, 0)),
            pl.BlockSpec((1, HID), lambda i: (0, 0)),
        ],
        out_specs=pl.BlockSpec((N // 10, HID), lambda i: (i, 0)),
        out_shape=jax.ShapeDtypeStruct((N, HID), jnp.float32),
    )(x, W1a, b1)


def _mlp_body(e1_ref, ea_ref, w1b_ref, w2_ref, b2_ref, w3_ref, b3_ref,
              w4_ref, b4_ref, m_ref):
    h1 = jnp.maximum(
        e1_ref[...]
        + jnp.dot(ea_ref[...], w1b_ref[...],
                  preferred_element_type=jnp.float32),
        0.0,
    ).astype(jnp.bfloat16)
    h2 = jnp.maximum(
        jnp.dot(h1, w2_ref[...], preferred_element_type=jnp.float32)
        + b2_ref[...],
        0.0,
    ).astype(jnp.bfloat16)
    h3 = jnp.maximum(
        jnp.dot(h2, w3_ref[...], preferred_element_type=jnp.float32)
        + b3_ref[...],
        0.0,
    ).astype(jnp.bfloat16)
    m_ref[...] = jnp.maximum(
        jnp.dot(h3, w4_ref[...], preferred_element_type=jnp.float32)
        + b4_ref[...],
        0.0,
    )


def _edge_mlp(e1, edge_attr, W1b, W2, b2, W3, b3, W4, b4):
    grid = (EC // MLP_BLOCK,)
    return pl.pallas_call(
        _mlp_body,
        grid=grid,
        in_specs=[
            pl.BlockSpec((MLP_BLOCK, HID), lambda i: (i, 0)),
            pl.BlockSpec((MLP_BLOCK, D_EDGE), lambda i: (i, 0)),
            pl.BlockSpec((D_EDGE, HID), lambda i: (0, 0)),
            pl.BlockSpec((HID, 2 * HID), lambda i: (0, 0)),
            pl.BlockSpec((1, 2 * HID), lambda i: (0, 0)),
            pl.BlockSpec((2 * HID, HID), lambda i: (0, 0)),
            pl.BlockSpec((1, HID), lambda i: (0, 0)),
            pl.BlockSpec((HID, D_OUT), lambda i: (0, 0)),
            pl.BlockSpec((1, D_OUT), lambda i: (0, 0)),
        ],
        out_specs=pl.BlockSpec((MLP_BLOCK, D_OUT), lambda i: (i, 0)),
        out_shape=jax.ShapeDtypeStruct((EC, D_OUT), jnp.float32),
    )(e1, edge_attr, W1b, W2, b2, W3, b3, W4, b4)


def _combine_body(sa_ref, sb_ref, c1, c2, c3, c4, c5, o_ref):
    total = sa_ref[0] + sa_ref[1] + sb_ref[0] + sb_ref[1]
    cnt = (jnp.sum(c1[...], axis=0) + jnp.sum(c2[...], axis=0)
           + jnp.sum(c3[...], axis=0) + jnp.sum(c4[...], axis=0)
           + jnp.sum(c5[...], axis=0))
    denom = jnp.maximum(cnt, 1.0)
    o_ref[...] = total / denom[:, None]


def _combine(sums_a, sums_b, cnts):
    grid = (10,)
    blk = NP // 10
    sspec = pl.BlockSpec((2, blk, D_OUT), lambda i: (0, i, 0))
    cspec = pl.BlockSpec((NW, blk), lambda i: (0, i))
    out = pl.pallas_call(
        _combine_body,
        grid=grid,
        in_specs=[sspec, sspec] + [cspec] * NCH,
        out_specs=pl.BlockSpec((blk, D_OUT), lambda i: (i, 0)),
        out_shape=jax.ShapeDtypeStruct((NP, D_OUT), jnp.float32),
    )(sums_a, sums_b, *cnts)
    return out[:N]


def kernel(x, edge_index, edge_attr, W1, b1, W2, b2, W3, b3, W4, b4):
    dst = edge_index[1].astype(jnp.int32)
    W1a = W1[:D_IN]
    W1b = W1[D_IN:].astype(jnp.bfloat16)
    W2b = W2.astype(jnp.bfloat16)
    W3b = W3.astype(jnp.bfloat16)
    W4b = W4.astype(jnp.bfloat16)
    ea = edge_attr.astype(jnp.bfloat16)
    b1r = b1.reshape(1, HID)
    b2r = b2.reshape(1, 2 * HID)
    b3r = b3.reshape(1, HID)
    b4r = b4.reshape(1, D_OUT)

    xp = _node_precompute(x, W1a, b1r)

    dst5 = dst.reshape(NCH, NW, NITER, KG)
    zeros = jnp.zeros((KG, HID), jnp.float32)

    ms = []
    cnts = []
    for ch in range(NCH):
        e1, cnt3 = _sc_gather(xp, dst5[ch])
        eac = lax.slice_in_dim(ea, ch * EC, (ch + 1) * EC, axis=0)
        m = _edge_mlp(e1, eac, W1b, W2b, b2r, W3b, b3r, W4b, b4r)
        ms.append(m)
        cnts.append(cnt3.reshape(NW, NP))

    sums_a = _sc_scatter2(ms[0], ms[1], dst5[0], dst5[1], zeros)
    sums_b = _sc_scatter3(ms[2], ms[3], ms[4], dst5[2], dst5[3], dst5[4],
                          zeros)
    return _combine(sums_a, sums_b, cnts)


# speedup vs baseline: 1.5748x; 1.5748x over previous
import functools

import jax
import jax.numpy as jnp
from jax import lax
from jax.experimental import pallas as pl
from jax.experimental.pallas import tpu as pltpu
from jax.experimental.pallas import tpu_sc as plsc

N = 10000
E = 320000
D_IN = 128
D_EDGE = 16
HID = 128
D_OUT = 128

MLP_BLOCK = 3200

NC = 2
NS = 16
NW = NC * NS

NCH = 5
EC = E // NCH
EPW = EC // NW
KG = 80
NITER = EPW // KG
NB = 5
NGRP = NITER // NB
NB_S = 4
NGRP_S = NITER // NB_S
NP = 10240
RPT = NP // NS
HROWS = 5
HR = 2048

_SC_MESH = plsc.VectorSubcoreMesh(
    core_axis_name="c", subcore_axis_name="s", num_cores=NC, num_subcores=NS
)
_SC_PARAMS = pltpu.CompilerParams(needs_layout_passes=False)


def _sc_gather_body(xp_hbm, dst_hbm, e1_hbm, cnt_hbm, idx_v, rows, hist_v,
                    sem_i, sems_g, sems_w):
    c = lax.axis_index("c")
    s = lax.axis_index("s")
    wid = c * NS + s
    base = wid * EPW

    idx_load = pltpu.async_copy(dst_hbm.at[wid], idx_v, sem_i)

    zeros16 = jnp.zeros((16,), jnp.float32)

    def zero_hist(i, carry):
        for r in range(HROWS):
            hist_v[r, pl.ds(i * 16, 16)] = zeros16
        return carry

    lax.fori_loop(0, HR // 16, zero_hist, 0)
    idx_load.wait()

    ones = jnp.ones((16,), jnp.float32)

    def hist_update(i):
        for j in range(KG // 16):
            idxs = idx_v[i, pl.ds(j * 16, 16)]
            plsc.addupdate_scatter(
                hist_v, [lax.shift_right_logical(idxs, 11),
                         lax.bitwise_and(idxs, HR - 1)], ones)

    def group(g, carry):
        descs = []
        for b in range(NB):
            i = g * NB + b

            @pl.when(g > 0)
            def _():
                pltpu.make_async_copy(
                    rows[b], e1_hbm.at[pl.ds(base + i * KG, KG)], sems_w[b]
                ).wait()

            descs.append(
                pltpu.async_copy(xp_hbm.at[idx_v.at[i]], rows[b], sems_g[b])
            )
        for b in range(NB):
            i = g * NB + b
            hist_update(i)
            descs[b].wait()
            pltpu.async_copy(rows[b], e1_hbm.at[pl.ds(base + i * KG, KG)],
                             sems_w[b])
        return carry

    lax.fori_loop(0, NGRP, group, 0)

    for b in range(NB):
        pltpu.make_async_copy(
            rows[b], e1_hbm.at[pl.ds(base, KG)], sems_w[b]
        ).wait()
    pltpu.sync_copy(hist_v, cnt_hbm.at[wid])


@functools.partial(
    pl.kernel,
    out_type=(
        jax.ShapeDtypeStruct((EC, HID), jnp.float32),
        jax.ShapeDtypeStruct((NW, HROWS, HR), jnp.float32),
    ),
    mesh=_SC_MESH,
    scratch_types=[
        pltpu.VMEM((NITER, KG), jnp.int32),
        [pltpu.VMEM((KG, HID), jnp.float32) for _ in range(NB)],
        pltpu.VMEM((HROWS, HR), jnp.float32),
        pltpu.SemaphoreType.DMA,
        [pltpu.SemaphoreType.DMA for _ in range(NB)],
        [pltpu.SemaphoreType.DMA for _ in range(NB)],
    ],
    compiler_params=_SC_PARAMS,
)
def _sc_gather(*refs):
    _sc_gather_body(*refs)


def _make_scatter(nm):
    def body(*refs):
        ms = refs[:nm]
        dsts = refs[nm:2 * nm]
        z_hbm = refs[2 * nm]
        sums_hbm = refs[2 * nm + 1]
        idx_v = refs[2 * nm + 2]
        rows = refs[2 * nm + 3]
        sem_i = refs[2 * nm + 4]
        sems_l = refs[2 * nm + 5]
        sems_s = refs[2 * nm + 6]
        acc_sh = refs[2 * nm + 7]

        c = lax.axis_index("c")
        s = lax.axis_index("s")
        wid = c * NS + s
        base = wid * EPW

        pltpu.sync_copy(z_hbm, rows[0])
        for k in range(RPT // KG):
            r0 = s * RPT + k * KG
            pltpu.sync_copy(rows[0], acc_sh.at[pl.ds(r0, KG)])
        plsc.subcore_barrier()

        for m_hbm, dst_hbm in zip(ms, dsts):
            pltpu.async_copy(dst_hbm.at[wid], idx_v, sem_i).wait()

            def group(g, carry):
                for b in range(NB_S):
                    i = g * NB_S + b

                    @pl.when(g > 0)
                    def _():
                        pltpu.make_async_copy(
                            rows[b], acc_sh.at[idx_v.at[i]], sems_s[b]
                        ).wait()

                    pltpu.async_copy(m_hbm.at[pl.ds(base + i * KG, KG)],
                                     rows[b], sems_l[b])
                for b in range(NB_S):
                    i = g * NB_S + b
                    pltpu.make_async_copy(
                        m_hbm.at[pl.ds(base, KG)], rows[b], sems_l[b]).wait()
                    pltpu.async_copy(rows[b], acc_sh.at[idx_v.at[i]],
                                     sems_s[b], add=True)
                return carry

            lax.fori_loop(0, NGRP_S, group, 0)

            for b in range(NB_S):
                pltpu.make_async_copy(
                    rows[b], acc_sh.at[idx_v.at[0]], sems_s[b]).wait()

            i = NGRP_S * NB_S
            pltpu.sync_copy(m_hbm.at[pl.ds(base + i * KG, KG)], rows[0])
            pltpu.sync_copy(rows[0], acc_sh.at[idx_v.at[i]], add=True)

        plsc.subcore_barrier()

        for k in range(RPT // KG):
            r0 = s * RPT + k * KG
            pltpu.sync_copy(acc_sh.at[pl.ds(r0, KG)], rows[0])
            pltpu.sync_copy(rows[0], sums_hbm.at[c, pl.ds(r0, KG)])

    return functools.partial(
        pl.kernel,
        out_type=jax.ShapeDtypeStruct((NC, NP, HID), jnp.float32),
        mesh=_SC_MESH,
        scratch_types=[
            pltpu.VMEM((NITER, KG), jnp.int32),
            [pltpu.VMEM((KG, HID), jnp.float32) for _ in range(NB_S)],
            pltpu.SemaphoreType.DMA,
            [pltpu.SemaphoreType.DMA for _ in range(NB_S)],
            [pltpu.SemaphoreType.DMA for _ in range(NB_S)],
            pltpu.VMEM_SHARED((NP, HID), jnp.float32),
        ],
        compiler_params=_SC_PARAMS,
    )(body)


_sc_scatter2 = _make_scatter(2)
_sc_scatter3 = _make_scatter(3)


def _xp_body(x_ref, w_ref, b_ref, o_ref):
    o_ref[...] = (
        jnp.dot(x_ref[...], w_ref[...], preferred_element_type=jnp.float32)
        + b_ref[...]
    )


def _node_precompute(x, W1a, b1):
    grid = (10,)
    return pl.pallas_call(
        _xp_body,
        grid=grid,
        in_specs=[
            pl.BlockSpec((N // 10, D_IN), lambda i: (i, 0)),
            pl.BlockSpec((D_IN, HID), lambda i: (0, 0)),
            pl.BlockSpec((1, HID), lambda i: (0, 0)),
        ],
        out_specs=pl.BlockSpec((N // 10, HID), lambda i: (i, 0)),
        out_shape=jax.ShapeDtypeStruct((N, HID), jnp.float32),
    )(x, W1a, b1)


def _mlp_body(e1_ref, ea_ref, w1b_ref, w2_ref, b2_ref, w3_ref, b3_ref,
              w4_ref, b4_ref, m_ref):
    eaw = lax.dot_general(
        ea_ref[...], w1b_ref[...], (((0,), (0,)), ((), ())),
        preferred_element_type=jnp.float32,
    )
    h1 = jnp.maximum(e1_ref[...] + eaw, 0.0).astype(jnp.bfloat16)
    h2 = jnp.maximum(
        jnp.dot(h1, w2_ref[...], preferred_element_type=jnp.float32)
        + b2_ref[...],
        0.0,
    ).astype(jnp.bfloat16)
    h3 = jnp.maximum(
        jnp.dot(h2, w3_ref[...], preferred_element_type=jnp.float32)
        + b3_ref[...],
        0.0,
    ).astype(jnp.bfloat16)
    m_ref[...] = jnp.maximum(
        jnp.dot(h3, w4_ref[...], preferred_element_type=jnp.float32)
        + b4_ref[...],
        0.0,
    )


def _edge_mlp(e1, eaT, ch, W1b, W2, b2, W3, b3, W4, b4):
    grid = (EC // MLP_BLOCK,)
    off = ch * (EC // MLP_BLOCK)
    return pl.pallas_call(
        _mlp_body,
        grid=grid,
        in_specs=[
            pl.BlockSpec((MLP_BLOCK, HID), lambda i: (i, 0)),
            pl.BlockSpec((D_EDGE, MLP_BLOCK), lambda i, o=off: (0, o + i)),
            pl.BlockSpec((D_EDGE, HID), lambda i: (0, 0)),
            pl.BlockSpec((HID, 2 * HID), lambda i: (0, 0)),
            pl.BlockSpec((1, 2 * HID), lambda i: (0, 0)),
            pl.BlockSpec((2 * HID, HID), lambda i: (0, 0)),
            pl.BlockSpec((1, HID), lambda i: (0, 0)),
            pl.BlockSpec((HID, D_OUT), lambda i: (0, 0)),
            pl.BlockSpec((1, D_OUT), lambda i: (0, 0)),
        ],
        out_specs=pl.BlockSpec((MLP_BLOCK, D_OUT), lambda i: (i, 0)),
        out_shape=jax.ShapeDtypeStruct((EC, D_OUT), jnp.float32),
    )(e1, eaT, W1b, W2, b2, W3, b3, W4, b4)


def _combine_body(sa_ref, sb_ref, c1, c2, c3, c4, c5, o_ref):
    total = sa_ref[0] + sa_ref[1] + sb_ref[0] + sb_ref[1]
    cnt = (jnp.sum(c1[...], axis=0) + jnp.sum(c2[...], axis=0)
           + jnp.sum(c3[...], axis=0) + jnp.sum(c4[...], axis=0)
           + jnp.sum(c5[...], axis=0))
    denom = jnp.maximum(cnt, 1.0)
    o_ref[...] = total / denom[:, None]


def _combine(sums_a, sums_b, cnts):
    grid = (10,)
    blk = NP // 10
    sspec = pl.BlockSpec((2, blk, D_OUT), lambda i: (0, i, 0))
    cspec = pl.BlockSpec((NW, blk), lambda i: (0, i))
    out = pl.pallas_call(
        _combine_body,
        grid=grid,
        in_specs=[sspec, sspec] + [cspec] * NCH,
        out_specs=pl.BlockSpec((blk, D_OUT), lambda i: (i, 0)),
        out_shape=jax.ShapeDtypeStruct((NP, D_OUT), jnp.float32),
    )(sums_a, sums_b, *cnts)
    return out[:N]


def kernel(x, edge_index, edge_attr, W1, b1, W2, b2, W3, b3, W4, b4):
    dst = edge_index[1].astype(jnp.int32)
    W1a = W1[:D_IN]
    W1b = W1[D_IN:]
    W2b = W2.astype(jnp.bfloat16)
    W3b = W3.astype(jnp.bfloat16)
    W4b = W4.astype(jnp.bfloat16)
    eaT = edge_attr.T
    b1r = b1.reshape(1, HID)
    b2r = b2.reshape(1, 2 * HID)
    b3r = b3.reshape(1, HID)
    b4r = b4.reshape(1, D_OUT)

    xp = _node_precompute(x, W1a, b1r)

    dst5 = dst.reshape(NCH, NW, NITER, KG)
    zeros = jnp.zeros((KG, HID), jnp.float32)

    ms = []
    cnts = []
    for ch in range(NCH):
        e1, cnt3 = _sc_gather(xp, dst5[ch])
        m = _edge_mlp(e1, eaT, ch, W1b, W2b, b2r, W3b, b3r, W4b, b4r)
        ms.append(m)
        cnts.append(cnt3.reshape(NW, NP))

    sums_a = _sc_scatter2(ms[0], ms[1], dst5[0], dst5[1], zeros)
    sums_b = _sc_scatter3(ms[2], ms[3], ms[4], dst5[2], dst5[3], dst5[4],
                          zeros)
    return _combine(sums_a, sums_b, cnts)


# speedup vs baseline: 1.5767x; 1.0012x over previous
import functools

import jax
import jax.numpy as jnp
from jax import lax
from jax.experimental import pallas as pl
from jax.experimental.pallas import tpu as pltpu
from jax.experimental.pallas import tpu_sc as plsc

N = 10000
E = 320000
D_IN = 128
D_EDGE = 16
HID = 128
D_OUT = 128

MLP_BLOCK = 3200

NC = 2
NS = 16
NW = NC * NS

NCH = 5
EC = E // NCH
EPW = EC // NW
KG = 80
NITER = EPW // KG
NB = 5
NGRP = NITER // NB
NB_S = 4
NGRP_S = NITER // NB_S
NP = 10240
RPT = NP // NS
HROWS = 5
HR = 2048

_SC_MESH = plsc.VectorSubcoreMesh(
    core_axis_name="c", subcore_axis_name="s", num_cores=NC, num_subcores=NS
)
_SC_PARAMS = pltpu.CompilerParams(needs_layout_passes=False)


def _sc_gather_body(xp_hbm, dst_hbm, e1_hbm, cnt_hbm, idx_v, rows, hist_v,
                    sem_i, sems_g, sems_w):
    c = lax.axis_index("c")
    s = lax.axis_index("s")
    wid = c * NS + s
    base = wid * EPW

    idx_load = pltpu.async_copy(dst_hbm.at[wid], idx_v, sem_i)

    zeros16 = jnp.zeros((16,), jnp.float32)

    def zero_hist(i, carry):
        for r in range(HROWS):
            hist_v[r, pl.ds(i * 16, 16)] = zeros16
        return carry

    lax.fori_loop(0, HR // 16, zero_hist, 0)
    idx_load.wait()

    ones = jnp.ones((16,), jnp.float32)

    def hist_update(i):
        for j in range(KG // 16):
            idxs = idx_v[i, pl.ds(j * 16, 16)]
            plsc.addupdate_scatter(
                hist_v, [lax.shift_right_logical(idxs, 11),
                         lax.bitwise_and(idxs, HR - 1)], ones)

    def group(g, carry):
        descs = []
        for b in range(NB):
            i = g * NB + b

            @pl.when(g > 0)
            def _():
                pltpu.make_async_copy(
                    rows[b], e1_hbm.at[pl.ds(base + i * KG, KG)], sems_w[b]
                ).wait()

            descs.append(
                pltpu.async_copy(xp_hbm.at[idx_v.at[i]], rows[b], sems_g[b])
            )
        for b in range(NB):
            i = g * NB + b
            hist_update(i)
            descs[b].wait()
            pltpu.async_copy(rows[b], e1_hbm.at[pl.ds(base + i * KG, KG)],
                             sems_w[b])
        return carry

    lax.fori_loop(0, NGRP, group, 0)

    for b in range(NB):
        pltpu.make_async_copy(
            rows[b], e1_hbm.at[pl.ds(base, KG)], sems_w[b]
        ).wait()
    pltpu.sync_copy(hist_v, cnt_hbm.at[wid])


@functools.partial(
    pl.kernel,
    out_type=(
        jax.ShapeDtypeStruct((EC, HID), jnp.float32),
        jax.ShapeDtypeStruct((NW, HROWS, HR), jnp.float32),
    ),
    mesh=_SC_MESH,
    scratch_types=[
        pltpu.VMEM((NITER, KG), jnp.int32),
        [pltpu.VMEM((KG, HID), jnp.float32) for _ in range(NB)],
        pltpu.VMEM((HROWS, HR), jnp.float32),
        pltpu.SemaphoreType.DMA,
        [pltpu.SemaphoreType.DMA for _ in range(NB)],
        [pltpu.SemaphoreType.DMA for _ in range(NB)],
    ],
    compiler_params=_SC_PARAMS,
)
def _sc_gather(*refs):
    _sc_gather_body(*refs)


def _make_scatter(nm):
    def body(*refs):
        ms = refs[:nm]
        dsts = refs[nm:2 * nm]
        z_hbm = refs[2 * nm]
        sums_hbm = refs[2 * nm + 1]
        idx_v = refs[2 * nm + 2]
        rows = refs[2 * nm + 3]
        sem_i = refs[2 * nm + 4]
        sems_l = refs[2 * nm + 5]
        sems_s = refs[2 * nm + 6]
        acc_sh = refs[2 * nm + 7]

        c = lax.axis_index("c")
        s = lax.axis_index("s")
        wid = c * NS + s
        base = wid * EPW

        pltpu.sync_copy(z_hbm, rows[0])
        for k in range(RPT // KG):
            r0 = s * RPT + k * KG
            pltpu.sync_copy(rows[0], acc_sh.at[pl.ds(r0, KG)])
        plsc.subcore_barrier()

        for m_hbm, dst_hbm in zip(ms, dsts):
            pltpu.async_copy(dst_hbm.at[wid], idx_v, sem_i).wait()

            def group(g, carry):
                for b in range(NB_S):
                    i = g * NB_S + b

                    @pl.when(g > 0)
                    def _():
                        pltpu.make_async_copy(
                            rows[b], acc_sh.at[idx_v.at[i]], sems_s[b]
                        ).wait()

                    pltpu.async_copy(m_hbm.at[pl.ds(base + i * KG, KG)],
                                     rows[b], sems_l[b])
                for b in range(NB_S):
                    i = g * NB_S + b
                    pltpu.make_async_copy(
                        m_hbm.at[pl.ds(base, KG)], rows[b], sems_l[b]).wait()
                    pltpu.async_copy(rows[b], acc_sh.at[idx_v.at[i]],
                                     sems_s[b], add=True)
                return carry

            lax.fori_loop(0, NGRP_S, group, 0)

            for b in range(NB_S):
                pltpu.make_async_copy(
                    rows[b], acc_sh.at[idx_v.at[0]], sems_s[b]).wait()

            i = NGRP_S * NB_S
            pltpu.sync_copy(m_hbm.at[pl.ds(base + i * KG, KG)], rows[0])
            pltpu.sync_copy(rows[0], acc_sh.at[idx_v.at[i]], add=True)

        plsc.subcore_barrier()

        for k in range(RPT // KG):
            r0 = s * RPT + k * KG
            pltpu.sync_copy(acc_sh.at[pl.ds(r0, KG)], rows[0])
            pltpu.sync_copy(rows[0], sums_hbm.at[c, pl.ds(r0, KG)])

    return functools.partial(
        pl.kernel,
        out_type=jax.ShapeDtypeStruct((NC, NP, HID), jnp.float32),
        mesh=_SC_MESH,
        scratch_types=[
            pltpu.VMEM((NITER, KG), jnp.int32),
            [pltpu.VMEM((KG, HID), jnp.float32) for _ in range(NB_S)],
            pltpu.SemaphoreType.DMA,
            [pltpu.SemaphoreType.DMA for _ in range(NB_S)],
            [pltpu.SemaphoreType.DMA for _ in range(NB_S)],
            pltpu.VMEM_SHARED((NP, HID), jnp.float32),
        ],
        compiler_params=_SC_PARAMS,
    )(body)


_sc_scatter2 = _make_scatter(2)
_sc_scatter3 = _make_scatter(3)


def _xp_body(x_ref, w_ref, b_ref, o_ref):
    o_ref[...] = (
        jnp.dot(x_ref[...], w_ref[...], preferred_element_type=jnp.float32)
        + b_ref[...]
    )


def _node_precompute(x, W1a, b1):
    grid = (10,)
    return pl.pallas_call(
        _xp_body,
        grid=grid,
        in_specs=[
            pl.BlockSpec((N // 10, D_IN), lambda i: (i, 0)),
            pl.BlockSpec((D_IN, HID), lambda i: (0, 0)),
            pl.BlockSpec((1, HID), lambda i: (0, 0)),
        ],
        out_specs=pl.BlockSpec((N // 10, HID), lambda i: (i, 0)),
        out_shape=jax.ShapeDtypeStruct((N, HID), jnp.float32),
    )(x, W1a, b1)


def _mlp_body(e1_ref, ea_ref, w1b_ref, w2_ref, b2_ref, w3_ref, b3_ref,
              w4_ref, b4_ref, m_ref):
    eaw = lax.dot_general(
        ea_ref[...], w1b_ref[...], (((0,), (0,)), ((), ())),
        preferred_element_type=jnp.float32,
    )
    h1 = jnp.maximum(e1_ref[...] + eaw, 0.0).astype(jnp.bfloat16)
    h2 = jnp.maximum(
        jnp.dot(h1, w2_ref[...], preferred_element_type=jnp.float32)
        + b2_ref[...],
        0.0,
    ).astype(jnp.bfloat16)
    h3 = jnp.maximum(
        jnp.dot(h2, w3_ref[...], preferred_element_type=jnp.float32)
        + b3_ref[...],
        0.0,
    ).astype(jnp.bfloat16)
    m_ref[...] = jnp.maximum(
        jnp.dot(h3, w4_ref[...], preferred_element_type=jnp.float32)
        + b4_ref[...],
        0.0,
    )


def _edge_mlp(e1, eaT, ch, W1b, W2, b2, W3, b3, W4, b4):
    grid = (EC // MLP_BLOCK,)
    off = ch * (EC // MLP_BLOCK)
    return pl.pallas_call(
        _mlp_body,
        grid=grid,
        in_specs=[
            pl.BlockSpec((MLP_BLOCK, HID), lambda i: (i, 0)),
            pl.BlockSpec((D_EDGE, MLP_BLOCK), lambda i, o=off: (0, o + i)),
            pl.BlockSpec((D_EDGE, HID), lambda i: (0, 0)),
            pl.BlockSpec((HID, 2 * HID), lambda i: (0, 0)),
            pl.BlockSpec((1, 2 * HID), lambda i: (0, 0)),
            pl.BlockSpec((2 * HID, HID), lambda i: (0, 0)),
            pl.BlockSpec((1, HID), lambda i: (0, 0)),
            pl.BlockSpec((HID, D_OUT), lambda i: (0, 0)),
            pl.BlockSpec((1, D_OUT), lambda i: (0, 0)),
        ],
        out_specs=pl.BlockSpec((MLP_BLOCK, D_OUT), lambda i: (i, 0)),
        out_shape=jax.ShapeDtypeStruct((EC, D_OUT), jnp.float32),
    )(e1, eaT, W1b, W2, b2, W3, b3, W4, b4)


def _combine_body(sa_ref, sb_ref, c1, c2, c3, c4, c5, o_ref):
    total = sa_ref[0] + sa_ref[1] + sb_ref[0] + sb_ref[1]
    cnt = (jnp.sum(c1[...], axis=0) + jnp.sum(c2[...], axis=0)
           + jnp.sum(c3[...], axis=0) + jnp.sum(c4[...], axis=0)
           + jnp.sum(c5[...], axis=0))
    denom = jnp.maximum(cnt, 1.0)
    o_ref[...] = total / denom[:, None]


def _combine(sums_a, sums_b, cnts):
    grid = (10,)
    blk = NP // 10
    sspec = pl.BlockSpec((2, blk, D_OUT), lambda i: (0, i, 0))
    cspec = pl.BlockSpec((NW, blk), lambda i: (0, i))
    out = pl.pallas_call(
        _combine_body,
        grid=grid,
        in_specs=[sspec, sspec] + [cspec] * NCH,
        out_specs=pl.BlockSpec((blk, D_OUT), lambda i: (i, 0)),
        out_shape=jax.ShapeDtypeStruct((NP, D_OUT), jnp.float32),
    )(sums_a, sums_b, *cnts)
    return out[:N]


def kernel(x, edge_index, edge_attr, W1, b1, W2, b2, W3, b3, W4, b4):
    dst = edge_index[1].astype(jnp.int32)
    W1a = W1[:D_IN]
    W1b = W1[D_IN:]
    W2b = W2.astype(jnp.bfloat16)
    W3b = W3.astype(jnp.bfloat16)
    W4b = W4.astype(jnp.bfloat16)
    eaT = edge_attr.T
    b1r = b1.reshape(1, HID)
    b2r = b2.reshape(1, 2 * HID)
    b3r = b3.reshape(1, HID)
    b4r = b4.reshape(1, D_OUT)

    xp = _node_precompute(x, W1a, b1r)

    dst5 = dst.reshape(NCH, NW, NITER, KG)
    zeros = jnp.zeros((KG, HID), jnp.float32)

    ms = []
    cnts = []
    for ch in range(NCH):
        e1, cnt3 = _sc_gather(xp, dst5[ch])
        m = _edge_mlp(e1, eaT, ch, W1b, W2b, b2r, W3b, b3r, W4b, b4r)
        ms.append(m)
        cnts.append(cnt3.reshape(NW, NP))

    sums_a = _sc_scatter3(ms[0], ms[1], ms[2], dst5[0], dst5[1], dst5[2],
                          zeros)
    sums_b = _sc_scatter2(ms[3], ms[4], dst5[3], dst5[4], zeros)
    return _combine(sums_a, sums_b, cnts)


# speedup vs baseline: 1.5933x; 1.0105x over previous
import functools

import jax
import jax.numpy as jnp
from jax import lax
from jax.experimental import pallas as pl
from jax.experimental.pallas import tpu as pltpu
from jax.experimental.pallas import tpu_sc as plsc

N = 10000
E = 320000
D_IN = 128
D_EDGE = 16
HID = 128
D_OUT = 128

MLP_BLOCK = 3200

NC = 2
NS = 16
NW = NC * NS

NCH = 5
EC = E // NCH
EPW = EC // NW
KG = 80
NITER = EPW // KG
NB = 2
NGRP = NITER // NB
NB_S = 4
NGRP_S = NITER // NB_S
NP = 10240
RPT = NP // NS
HROWS = 5
HR = 2048

_SC_MESH = plsc.VectorSubcoreMesh(
    core_axis_name="c", subcore_axis_name="s", num_cores=NC, num_subcores=NS
)
_SC_PARAMS = pltpu.CompilerParams(needs_layout_passes=False)


def _sc_gather_body(xp_hbm, dst_hbm, e1_hbm, cnt_hbm, idx_v, rows, hist_v,
                    xp_sh, sem_i, sems_g, sems_w):
    c = lax.axis_index("c")
    s = lax.axis_index("s")
    wid = c * NS + s
    base = wid * EPW

    idx_load = pltpu.async_copy(dst_hbm.at[wid], idx_v, sem_i)

    for k in range(RPT // KG):
        r0 = s * RPT + k * KG
        pltpu.sync_copy(xp_hbm.at[pl.ds(r0, KG)], rows[0])
        pltpu.sync_copy(rows[0], xp_sh.at[pl.ds(r0, KG)])

    zeros16 = jnp.zeros((16,), jnp.float32)

    def zero_hist(i, carry):
        for r in range(HROWS):
            hist_v[r, pl.ds(i * 16, 16)] = zeros16
        return carry

    lax.fori_loop(0, HR // 16, zero_hist, 0)
    idx_load.wait()
    plsc.subcore_barrier()

    ones = jnp.ones((16,), jnp.float32)

    def hist_update(i):
        for j in range(KG // 16):
            idxs = idx_v[i, pl.ds(j * 16, 16)]
            plsc.addupdate_scatter(
                hist_v, [lax.shift_right_logical(idxs, 11),
                         lax.bitwise_and(idxs, HR - 1)], ones)

    def group(g, carry):
        descs = []
        for b in range(NB):
            i = g * NB + b

            @pl.when(g > 0)
            def _():
                pltpu.make_async_copy(
                    rows[b], e1_hbm.at[pl.ds(base + i * KG, KG)], sems_w[b]
                ).wait()

            descs.append(
                pltpu.async_copy(xp_sh.at[idx_v.at[i]], rows[b], sems_g[b])
            )
        for b in range(NB):
            i = g * NB + b
            hist_update(i)
            descs[b].wait()
            pltpu.async_copy(rows[b], e1_hbm.at[pl.ds(base + i * KG, KG)],
                             sems_w[b])
        return carry

    lax.fori_loop(0, NGRP, group, 0)

    for b in range(NB):
        pltpu.make_async_copy(
            rows[b], e1_hbm.at[pl.ds(base, KG)], sems_w[b]
        ).wait()

    i_ep = NGRP * NB
    hist_update(i_ep)
    pltpu.async_copy(xp_sh.at[idx_v.at[i_ep]], rows[0], sems_g[0]).wait()
    pltpu.sync_copy(rows[0], e1_hbm.at[pl.ds(base + i_ep * KG, KG)])
    pltpu.sync_copy(hist_v, cnt_hbm.at[wid])


@functools.partial(
    pl.kernel,
    out_type=(
        jax.ShapeDtypeStruct((EC, HID), jnp.float32),
        jax.ShapeDtypeStruct((NW, HROWS, HR), jnp.float32),
    ),
    mesh=_SC_MESH,
    scratch_types=[
        pltpu.VMEM((NITER, KG), jnp.int32),
        [pltpu.VMEM((KG, HID), jnp.float32) for _ in range(NB)],
        pltpu.VMEM((HROWS, HR), jnp.float32),
        pltpu.VMEM_SHARED((NP, HID), jnp.float32),
        pltpu.SemaphoreType.DMA,
        [pltpu.SemaphoreType.DMA for _ in range(NB)],
        [pltpu.SemaphoreType.DMA for _ in range(NB)],
    ],
    compiler_params=_SC_PARAMS,
)
def _sc_gather(*refs):
    _sc_gather_body(*refs)


def _make_scatter(nm):
    def body(*refs):
        ms = refs[:nm]
        dsts = refs[nm:2 * nm]
        z_hbm = refs[2 * nm]
        sums_hbm = refs[2 * nm + 1]
        idx_v = refs[2 * nm + 2]
        rows = refs[2 * nm + 3]
        sem_i = refs[2 * nm + 4]
        sems_l = refs[2 * nm + 5]
        sems_s = refs[2 * nm + 6]
        acc_sh = refs[2 * nm + 7]

        c = lax.axis_index("c")
        s = lax.axis_index("s")
        wid = c * NS + s
        base = wid * EPW

        pltpu.sync_copy(z_hbm, rows[0])
        for k in range(RPT // KG):
            r0 = s * RPT + k * KG
            pltpu.sync_copy(rows[0], acc_sh.at[pl.ds(r0, KG)])
        plsc.subcore_barrier()

        for m_hbm, dst_hbm in zip(ms, dsts):
            pltpu.async_copy(dst_hbm.at[wid], idx_v, sem_i).wait()

            def group(g, carry):
                for b in range(NB_S):
                    i = g * NB_S + b

                    @pl.when(g > 0)
                    def _():
                        pltpu.make_async_copy(
                            rows[b], acc_sh.at[idx_v.at[i]], sems_s[b]
                        ).wait()

                    pltpu.async_copy(m_hbm.at[pl.ds(base + i * KG, KG)],
                                     rows[b], sems_l[b])
                for b in range(NB_S):
                    i = g * NB_S + b
                    pltpu.make_async_copy(
                        m_hbm.at[pl.ds(base, KG)], rows[b], sems_l[b]).wait()
                    pltpu.async_copy(rows[b], acc_sh.at[idx_v.at[i]],
                                     sems_s[b], add=True)
                return carry

            lax.fori_loop(0, NGRP_S, group, 0)

            for b in range(NB_S):
                pltpu.make_async_copy(
                    rows[b], acc_sh.at[idx_v.at[0]], sems_s[b]).wait()

            i = NGRP_S * NB_S
            pltpu.sync_copy(m_hbm.at[pl.ds(base + i * KG, KG)], rows[0])
            pltpu.sync_copy(rows[0], acc_sh.at[idx_v.at[i]], add=True)

        plsc.subcore_barrier()

        for k in range(RPT // KG):
            r0 = s * RPT + k * KG
            pltpu.sync_copy(acc_sh.at[pl.ds(r0, KG)], rows[0])
            pltpu.sync_copy(rows[0], sums_hbm.at[c, pl.ds(r0, KG)])

    return functools.partial(
        pl.kernel,
        out_type=jax.ShapeDtypeStruct((NC, NP, HID), jnp.float32),
        mesh=_SC_MESH,
        scratch_types=[
            pltpu.VMEM((NITER, KG), jnp.int32),
            [pltpu.VMEM((KG, HID), jnp.float32) for _ in range(NB_S)],
            pltpu.SemaphoreType.DMA,
            [pltpu.SemaphoreType.DMA for _ in range(NB_S)],
            [pltpu.SemaphoreType.DMA for _ in range(NB_S)],
            pltpu.VMEM_SHARED((NP, HID), jnp.float32),
        ],
        compiler_params=_SC_PARAMS,
    )(body)


_sc_scatter2 = _make_scatter(2)
_sc_scatter3 = _make_scatter(3)


def _xp_body(x_ref, w_ref, b_ref, o_ref):
    o_ref[...] = (
        jnp.dot(x_ref[...], w_ref[...], preferred_element_type=jnp.float32)
        + b_ref[...]
    )


def _node_precompute(x, W1a, b1):
    grid = (10,)
    return pl.pallas_call(
        _xp_body,
        grid=grid,
        in_specs=[
            pl.BlockSpec((NP // 10, D_IN), lambda i: (i, 0)),
            pl.BlockSpec((D_IN, HID), lambda i: (0, 0)),
            pl.BlockSpec((1, HID), lambda i: (0, 0)),
        ],
        out_specs=pl.BlockSpec((NP // 10, HID), lambda i: (i, 0)),
        out_shape=jax.ShapeDtypeStruct((NP, HID), jnp.float32),
    )(x, W1a, b1)


def _mlp_body(e1_ref, ea_ref, w1b_ref, w2_ref, b2_ref, w3_ref, b3_ref,
              w4_ref, b4_ref, m_ref):
    eaw = lax.dot_general(
        ea_ref[...], w1b_ref[...], (((0,), (0,)), ((), ())),
        preferred_element_type=jnp.float32,
    )
    h1 = jnp.maximum(e1_ref[...] + eaw, 0.0).astype(jnp.bfloat16)
    h2 = jnp.maximum(
        jnp.dot(h1, w2_ref[...], preferred_element_type=jnp.float32)
        + b2_ref[...],
        0.0,
    ).astype(jnp.bfloat16)
    h3 = jnp.maximum(
        jnp.dot(h2, w3_ref[...], preferred_element_type=jnp.float32)
        + b3_ref[...],
        0.0,
    ).astype(jnp.bfloat16)
    m_ref[...] = jnp.maximum(
        jnp.dot(h3, w4_ref[...], preferred_element_type=jnp.float32)
        + b4_ref[...],
        0.0,
    )


def _edge_mlp(e1, eaT, ch, W1b, W2, b2, W3, b3, W4, b4):
    grid = (EC // MLP_BLOCK,)
    off = ch * (EC // MLP_BLOCK)
    return pl.pallas_call(
        _mlp_body,
        grid=grid,
        in_specs=[
            pl.BlockSpec((MLP_BLOCK, HID), lambda i: (i, 0)),
            pl.BlockSpec((D_EDGE, MLP_BLOCK), lambda i, o=off: (0, o + i)),
            pl.BlockSpec((D_EDGE, HID), lambda i: (0, 0)),
            pl.BlockSpec((HID, 2 * HID), lambda i: (0, 0)),
            pl.BlockSpec((1, 2 * HID), lambda i: (0, 0)),
            pl.BlockSpec((2 * HID, HID), lambda i: (0, 0)),
            pl.BlockSpec((1, HID), lambda i: (0, 0)),
            pl.BlockSpec((HID, D_OUT), lambda i: (0, 0)),
            pl.BlockSpec((1, D_OUT), lambda i: (0, 0)),
        ],
        out_specs=pl.BlockSpec((MLP_BLOCK, D_OUT), lambda i: (i, 0)),
        out_shape=jax.ShapeDtypeStruct((EC, D_OUT), jnp.float32),
    )(e1, eaT, W1b, W2, b2, W3, b3, W4, b4)


def _combine_body(sa_ref, sb_ref, c1, c2, c3, c4, c5, o_ref):
    total = sa_ref[0] + sa_ref[1] + sb_ref[0] + sb_ref[1]
    cnt = (jnp.sum(c1[...], axis=0) + jnp.sum(c2[...], axis=0)
           + jnp.sum(c3[...], axis=0) + jnp.sum(c4[...], axis=0)
           + jnp.sum(c5[...], axis=0))
    denom = jnp.maximum(cnt, 1.0)
    o_ref[...] = total / denom[:, None]


def _combine(sums_a, sums_b, cnts):
    grid = (10,)
    blk = NP // 10
    sspec = pl.BlockSpec((2, blk, D_OUT), lambda i: (0, i, 0))
    cspec = pl.BlockSpec((NW, blk), lambda i: (0, i))
    out = pl.pallas_call(
        _combine_body,
        grid=grid,
        in_specs=[sspec, sspec] + [cspec] * NCH,
        out_specs=pl.BlockSpec((blk, D_OUT), lambda i: (i, 0)),
        out_shape=jax.ShapeDtypeStruct((NP, D_OUT), jnp.float32),
    )(sums_a, sums_b, *cnts)
    return out[:N]


def kernel(x, edge_index, edge_attr, W1, b1, W2, b2, W3, b3, W4, b4):
    dst = edge_index[1].astype(jnp.int32)
    W1a = W1[:D_IN]
    W1b = W1[D_IN:]
    W2b = W2.astype(jnp.bfloat16)
    W3b = W3.astype(jnp.bfloat16)
    W4b = W4.astype(jnp.bfloat16)
    eaT = edge_attr.T
    b1r = b1.reshape(1, HID)
    b2r = b2.reshape(1, 2 * HID)
    b3r = b3.reshape(1, HID)
    b4r = b4.reshape(1, D_OUT)

    xp = _node_precompute(x, W1a, b1r)

    dst5 = dst.reshape(NCH, NW, NITER, KG)
    zeros = jnp.zeros((KG, HID), jnp.float32)

    ms = []
    cnts = []
    for ch in range(NCH):
        e1, cnt3 = _sc_gather(xp, dst5[ch])
        m = _edge_mlp(e1, eaT, ch, W1b, W2b, b2r, W3b, b3r, W4b, b4r)
        ms.append(m)
        cnts.append(cnt3.reshape(NW, NP))

    sums_a = _sc_scatter3(ms[0], ms[1], ms[2], dst5[0], dst5[1], dst5[2],
                          zeros)
    sums_b = _sc_scatter2(ms[3], ms[4], dst5[3], dst5[4], zeros)
    return _combine(sums_a, sums_b, cnts)
